# baseline (speedup 1.0000x reference)
import functools

import jax
import jax.numpy as jnp
from jax import lax
from jax.experimental import pallas as pl
from jax.experimental.pallas import tpu as pltpu
from jax.experimental.pallas import tpu_sc as plsc

NC = 2
NS = 16
LN = 16


def _mesh():
    return plsc.VectorSubcoreMesh(
        core_axis_name="c", subcore_axis_name="s", num_cores=NC, num_subcores=NS
    )


def _deg_body(N, E, CH, NR, ef_hbm, out_hbm, acc, zbuf, obuf, ci, *ss):
    c = lax.axis_index("c")
    s = lax.axis_index("s")
    ct = N // 1000
    r0 = s * 1000

    def zrow(i, carry):
        zbuf[i, pl.ds(0, LN)] = jnp.zeros((LN,), jnp.float32)
        return carry

    lax.fori_loop(0, zbuf.shape[0], zrow, 0)

    def orow(i, carry):
        obuf[i, pl.ds(0, LN)] = jnp.ones((LN,), jnp.float32)
        return carry

    lax.fori_loop(0, CH, orow, 0)

    ec = E // NC
    et = ec // NS
    ni = et // CH
    base0 = c * ec + s * et
    pltpu.async_copy(ef_hbm.at[pl.ds(E + base0, et)], ci, ss[0])

    @pl.when(s < ct)
    def _zero():
        pltpu.sync_copy(zbuf, acc.at[pl.ds(r0, 1000)])

    pltpu.make_async_copy(ef_hbm.at[pl.ds(E + base0, et)], ci, ss[0]).wait()
    plsc.subcore_barrier()

    def start_scatter(b, ch):
        pltpu.async_copy(obuf, acc.at[ci.at[pl.ds(ch * CH, CH)]], ss[b], add=True)

    def wait_scatter(b):
        pltpu.make_async_copy(obuf, acc.at[ci.at[pl.ds(0, CH)]], ss[b]).wait()

    def outer(o, carry):
        for b in range(NR):
            ch = o * NR + b

            @pl.when(ch >= NR)
            def _drain():
                wait_scatter(b)

            start_scatter(b, ch)
        return carry

    lax.fori_loop(0, ni // NR, outer, 0)
    for ch in range((ni // NR) * NR, ni):
        b = ch % NR
        if ch >= NR:
            wait_scatter(b)
        start_scatter(b, ch)
    for b in range(NR):
        wait_scatter(b)

    plsc.subcore_barrier()

    @pl.when(s < ct)
    def _copyout():
        pltpu.sync_copy(acc.at[pl.ds(r0, 1000)], out_hbm.at[pl.ds(c * N + r0, 1000)])


def _sc_deg(ef, N, E, CH):
    NR = 4
    et = E // NC // NS
    body = functools.partial(_deg_body, N, E, CH, NR)
    return pl.kernel(
        body,
        out_type=jax.ShapeDtypeStruct((NC * N, LN), jnp.float32),
        mesh=_mesh(),
        compiler_params=pltpu.CompilerParams(use_tc_tiling_on_sc=False),
        scratch_types=[
            pltpu.VMEM_SHARED((N, LN), jnp.float32),
            pltpu.VMEM((1000, LN), jnp.float32),
            pltpu.VMEM((CH, LN), jnp.float32),
            pltpu.VMEM((et,), jnp.int32),
        ]
        + [pltpu.SemaphoreType.DMA] * NR,
    )(ef)


NB = 3


def _agg_body(N, E, D, CH, vals_hbm, ef_hbm, z_hbm, out_hbm, acc, ri, ci, *bufs):
    gb = bufs[0:NB]
    sm = bufs[NB:2 * NB]
    ss = bufs[2 * NB:3 * NB]
    c = lax.axis_index("c")
    s = lax.axis_index("s")
    ct = N // 1000
    r0 = s * 1000

    ec = E // NC
    et = ec // NS
    ni = et // CH
    base0 = c * ec + s * et

    pltpu.async_copy(ef_hbm.at[pl.ds(base0, et)], ri, sm[0])
    pltpu.async_copy(ef_hbm.at[pl.ds(E + base0, et)], ci, sm[1])

    @pl.when(s < ct)
    def _zero():
        pltpu.sync_copy(z_hbm, acc.at[pl.ds(r0, 1000)])

    pltpu.make_async_copy(ef_hbm.at[pl.ds(base0, et)], ri, sm[0]).wait()
    pltpu.make_async_copy(ef_hbm.at[pl.ds(E + base0, et)], ci, sm[1]).wait()
    plsc.subcore_barrier()

    def start_gather(b, ch):
        pltpu.async_copy(vals_hbm.at[ri.at[pl.ds(ch * CH, CH)]], gb[b], sm[b])

    def wait_gather(b):
        pltpu.make_async_copy(vals_hbm.at[ri.at[pl.ds(0, CH)]], gb[b], sm[b]).wait()

    def start_scatter(b, ch):
        pltpu.async_copy(gb[b], acc.at[ci.at[pl.ds(ch * CH, CH)]], ss[b], add=True)

    def wait_scatter(b):
        pltpu.make_async_copy(gb[b], acc.at[ci.at[pl.ds(0, CH)]], ss[b]).wait()

    for b in range(NB - 1):
        start_gather(b, b)

    def outer(o, carry):
        for b in range(NB):
            ch = o * NB + b
            pb = (b + NB - 1) % NB
            wait_gather(b)
            start_scatter(b, ch)

            @pl.when(ch + NB - 1 < ni)
            def _prefetch():
                @pl.when(ch >= 1)
                def _drain():
                    wait_scatter(pb)

                start_gather(pb, ch + NB - 1)
        return carry

    lax.fori_loop(0, ni // NB, outer, 0)
    for ch in range((ni // NB) * NB, ni):
        b = ch % NB
        pb = (b + NB - 1) % NB
        wait_gather(b)
        start_scatter(b, ch)
        if ch + NB - 1 < ni:
            wait_scatter(pb)
            start_gather(pb, ch + NB - 1)
    for b in range(NB):
        wait_scatter(b)

    plsc.subcore_barrier()

    @pl.when(s < ct)
    def _copyout():
        pltpu.sync_copy(acc.at[pl.ds(r0, 1000)], out_hbm.at[pl.ds(c * N + r0, 1000)])


def _sc_agg(vals, ef, zb, N, E, D, CH):
    et = E // NC // NS
    body = functools.partial(_agg_body, N, E, D, CH)
    return pl.kernel(
        body,
        out_type=jax.ShapeDtypeStruct((NC * N, D), jnp.float32),
        mesh=_mesh(),
        compiler_params=pltpu.CompilerParams(use_tc_tiling_on_sc=False),
        scratch_types=[
            pltpu.VMEM_SHARED((N, D), jnp.float32),
            pltpu.VMEM((et,), jnp.int32),
            pltpu.VMEM((et,), jnp.int32),
        ]
        + [pltpu.VMEM((CH, D), jnp.float32)] * NB
        + [pltpu.SemaphoreType.DMA] * (2 * NB),
    )(vals, ef, zb)


def _prep_body(deg0_ref, deg1_ref, x_ref, xs_ref, dv_ref):
    deg = deg0_ref[:, 0:1] + deg1_ref[:, 0:1]
    dinv = jnp.where(deg > 0, lax.rsqrt(deg), 0.0)
    dv = jnp.broadcast_to(dinv, x_ref.shape)
    dv_ref[...] = dv
    xs_ref[...] = x_ref[...] * dv


def _tc_prep(degp, x, N, D, R):
    nb = N // R
    return pl.pallas_call(
        _prep_body,
        grid=(nb,),
        in_specs=[
            pl.BlockSpec((R, LN), lambda i: (i, 0)),
            pl.BlockSpec((R, LN), lambda i: (nb + i, 0)),
            pl.BlockSpec((R, D), lambda i: (i, 0)),
        ],
        out_specs=[
            pl.BlockSpec((R, D), lambda i: (i, 0)),
            pl.BlockSpec((R, D), lambda i: (i, 0)),
        ],
        out_shape=[
            jax.ShapeDtypeStruct((N, D), jnp.float32),
            jax.ShapeDtypeStruct((N, D), jnp.float32),
        ],
    )(degp, degp, x)


def _mid_body(p0_ref, p1_ref, dv_ref, w1_ref, b1_ref, w2_ref, t_ref):
    dv = dv_ref[...]
    z = (p0_ref[...] + p1_ref[...]) * dv
    h = jnp.dot(z, w1_ref[...], preferred_element_type=jnp.float32) + b1_ref[...]
    h = jnp.maximum(h, 0.0)
    t_ref[...] = jnp.dot(h, w2_ref[...], preferred_element_type=jnp.float32) * dv


def _tc_mid(p, dv, W1, b1, W2, N, D, H, R):
    nb = N // R
    return pl.pallas_call(
        _mid_body,
        grid=(nb,),
        in_specs=[
            pl.BlockSpec((R, D), lambda i: (i, 0)),
            pl.BlockSpec((R, D), lambda i: (nb + i, 0)),
            pl.BlockSpec((R, D), lambda i: (i, 0)),
            pl.BlockSpec((D, H), lambda i: (0, 0)),
            pl.BlockSpec((1, H), lambda i: (0, 0)),
            pl.BlockSpec((H, D), lambda i: (0, 0)),
        ],
        out_specs=pl.BlockSpec((R, D), lambda i: (i, 0)),
        out_shape=jax.ShapeDtypeStruct((N, D), jnp.float32),
    )(p, p, dv, W1, b1.reshape(1, H), W2)


def _fin_body(q0_ref, q1_ref, dv_ref, b2_ref, o_ref):
    o_ref[...] = (q0_ref[...] + q1_ref[...]) * dv_ref[...] + b2_ref[...]


def _tc_fin(q, dv, b2, N, D, R):
    nb = N // R
    return pl.pallas_call(
        _fin_body,
        grid=(nb,),
        in_specs=[
            pl.BlockSpec((R, D), lambda i: (i, 0)),
            pl.BlockSpec((R, D), lambda i: (nb + i, 0)),
            pl.BlockSpec((R, D), lambda i: (i, 0)),
            pl.BlockSpec((1, D), lambda i: (0, 0)),
        ],
        out_specs=pl.BlockSpec((R, D), lambda i: (i, 0)),
        out_shape=jax.ShapeDtypeStruct((N, D), jnp.float32),
    )(q, q, dv, b2.reshape(1, D))


def kernel(x, edge_index, W1, b1, W2, b2):
    N, D = x.shape
    E = edge_index.shape[1]
    H = W1.shape[1]
    CH = 80
    CHA = 80
    R = 10000

    ef = edge_index.reshape(-1)
    zb = jnp.zeros((1000, D), jnp.float32)
    degp = _sc_deg(ef, N, E, CH)
    xs, dv = _tc_prep(degp, x, N, D, R)
    p = _sc_agg(xs, ef, zb, N, E, D, CHA)
    t = _tc_mid(p, dv, W1, b1, W2, N, D, H, R)
    q = _sc_agg(t, ef, zb, N, E, D, CHA)
    return _tc_fin(q, dv, b2, N, D, R)


# speedup vs baseline: 1.0146x; 1.0146x over previous
import functools

import jax
import jax.numpy as jnp
from jax import lax
from jax.experimental import pallas as pl
from jax.experimental.pallas import tpu as pltpu
from jax.experimental.pallas import tpu_sc as plsc

NC = 2
NS = 16
LN = 16


def _mesh():
    return plsc.VectorSubcoreMesh(
        core_axis_name="c", subcore_axis_name="s", num_cores=NC, num_subcores=NS
    )


def _deg_body(N, E, CH, NR, ef_hbm, out_hbm, acc, zbuf, obuf, ci, *ss):
    c = lax.axis_index("c")
    s = lax.axis_index("s")
    ct = N // 1000
    r0 = s * 1000

    def zrow(i, carry):
        zbuf[i, pl.ds(0, LN)] = jnp.zeros((LN,), jnp.float32)
        return carry

    lax.fori_loop(0, zbuf.shape[0], zrow, 0)

    def orow(i, carry):
        obuf[i, pl.ds(0, LN)] = jnp.ones((LN,), jnp.float32)
        return carry

    lax.fori_loop(0, CH, orow, 0)

    ec = E // NC
    et = ec // NS
    ni = et // CH
    base0 = c * ec + s * et
    pltpu.async_copy(ef_hbm.at[pl.ds(E + base0, et)], ci, ss[0])

    @pl.when(s < ct)
    def _zero():
        pltpu.sync_copy(zbuf, acc.at[pl.ds(r0, 1000)])

    pltpu.make_async_copy(ef_hbm.at[pl.ds(E + base0, et)], ci, ss[0]).wait()
    plsc.subcore_barrier()

    def start_scatter(b, ch):
        pltpu.async_copy(obuf, acc.at[ci.at[pl.ds(ch * CH, CH)]], ss[b], add=True)

    def wait_scatter(b):
        pltpu.make_async_copy(obuf, acc.at[ci.at[pl.ds(0, CH)]], ss[b]).wait()

    def outer(o, carry):
        for b in range(NR):
            ch = o * NR + b

            @pl.when(ch >= NR)
            def _drain():
                wait_scatter(b)

            start_scatter(b, ch)
        return carry

    lax.fori_loop(0, ni // NR, outer, 0)
    for ch in range((ni // NR) * NR, ni):
        b = ch % NR
        if ch >= NR:
            wait_scatter(b)
        start_scatter(b, ch)
    for b in range(NR):
        wait_scatter(b)

    plsc.subcore_barrier()

    @pl.when(s < ct)
    def _copyout():
        pltpu.sync_copy(acc.at[pl.ds(r0, 1000)], out_hbm.at[pl.ds(c * N + r0, 1000)])


def _sc_deg(ef, N, E, CH):
    NR = 4
    et = E // NC // NS
    body = functools.partial(_deg_body, N, E, CH, NR)
    return pl.kernel(
        body,
        out_type=jax.ShapeDtypeStruct((NC * N, LN), jnp.float32),
        mesh=_mesh(),
        compiler_params=pltpu.CompilerParams(use_tc_tiling_on_sc=False),
        scratch_types=[
            pltpu.VMEM_SHARED((N, LN), jnp.float32),
            pltpu.VMEM((1000, LN), jnp.float32),
            pltpu.VMEM((CH, LN), jnp.float32),
            pltpu.VMEM((et,), jnp.int32),
        ]
        + [pltpu.SemaphoreType.DMA] * NR,
    )(ef)


NB = 3


def _agg_body(N, E, D, CH, vals_hbm, ef_hbm, z_hbm, out_hbm, acc, ri, ci, *bufs):
    gb = bufs[0:NB]
    sm = bufs[NB:2 * NB]
    ss = bufs[2 * NB:3 * NB]
    c = lax.axis_index("c")
    s = lax.axis_index("s")
    ct = N // 1000
    r0 = s * 1000

    ec = E // NC
    et = ec // NS
    ni = et // CH
    base0 = c * ec + s * et

    pltpu.async_copy(ef_hbm.at[pl.ds(base0, et)], ri, sm[0])
    pltpu.async_copy(ef_hbm.at[pl.ds(E + base0, et)], ci, sm[1])

    @pl.when(s < ct)
    def _zero():
        pltpu.sync_copy(z_hbm, acc.at[pl.ds(r0, 1000)])

    pltpu.make_async_copy(ef_hbm.at[pl.ds(base0, et)], ri, sm[0]).wait()
    pltpu.make_async_copy(ef_hbm.at[pl.ds(E + base0, et)], ci, sm[1]).wait()
    plsc.subcore_barrier()

    def start_gather(b, ch):
        pltpu.async_copy(vals_hbm.at[ri.at[pl.ds(ch * CH, CH)]], gb[b], sm[b])

    def wait_gather(b):
        pltpu.make_async_copy(vals_hbm.at[ri.at[pl.ds(0, CH)]], gb[b], sm[b]).wait()

    def start_scatter(b, ch):
        pltpu.async_copy(gb[b], acc.at[ci.at[pl.ds(ch * CH, CH)]], ss[b], add=True)

    def wait_scatter(b):
        pltpu.make_async_copy(gb[b], acc.at[ci.at[pl.ds(0, CH)]], ss[b]).wait()

    for b in range(NB - 1):
        start_gather(b, b)

    def outer(o, carry):
        for b in range(NB):
            ch = o * NB + b
            pb = (b + NB - 1) % NB
            wait_gather(b)
            start_scatter(b, ch)

            @pl.when(ch + NB - 1 < ni)
            def _prefetch():
                @pl.when(ch >= 1)
                def _drain():
                    wait_scatter(pb)

                start_gather(pb, ch + NB - 1)
        return carry

    lax.fori_loop(0, ni // NB, outer, 0)
    for ch in range((ni // NB) * NB, ni):
        b = ch % NB
        pb = (b + NB - 1) % NB
        wait_gather(b)
        start_scatter(b, ch)
        if ch + NB - 1 < ni:
            wait_scatter(pb)
            start_gather(pb, ch + NB - 1)
    for b in range(NB):
        wait_scatter(b)

    plsc.subcore_barrier()

    @pl.when(s < ct)
    def _copyout():
        pltpu.sync_copy(acc.at[pl.ds(r0, 1000)], out_hbm.at[pl.ds(c * N + r0, 1000)])


def _sc_agg(vals, ef, zb, N, E, D, CH):
    et = E // NC // NS
    body = functools.partial(_agg_body, N, E, D, CH)
    return pl.kernel(
        body,
        out_type=jax.ShapeDtypeStruct((NC * N, D), jnp.float32),
        mesh=_mesh(),
        compiler_params=pltpu.CompilerParams(use_tc_tiling_on_sc=False),
        scratch_types=[
            pltpu.VMEM_SHARED((N, D), jnp.float32),
            pltpu.VMEM((et,), jnp.int32),
            pltpu.VMEM((et,), jnp.int32),
        ]
        + [pltpu.VMEM((CH, D), jnp.float32)] * NB
        + [pltpu.SemaphoreType.DMA] * (2 * NB),
    )(vals, ef, zb)


def _prep_body(deg0_ref, deg1_ref, x_ref, xs_ref, dv_ref):
    deg = deg0_ref[:, 0:1] + deg1_ref[:, 0:1]
    dinv = jnp.where(deg > 0, lax.rsqrt(deg), 0.0)
    dv = jnp.broadcast_to(dinv, x_ref.shape)
    dv_ref[...] = dv
    xs_ref[...] = x_ref[...] * dv


def _tc_prep(degp, x, N, D, R):
    nb = N // R
    return pl.pallas_call(
        _prep_body,
        grid=(nb,),
        in_specs=[
            pl.BlockSpec((R, LN), lambda i: (i, 0)),
            pl.BlockSpec((R, LN), lambda i: (nb + i, 0)),
            pl.BlockSpec((R, D), lambda i: (i, 0)),
        ],
        out_specs=[
            pl.BlockSpec((R, D), lambda i: (i, 0)),
            pl.BlockSpec((R, D), lambda i: (i, 0)),
        ],
        out_shape=[
            jax.ShapeDtypeStruct((N, D), jnp.float32),
            jax.ShapeDtypeStruct((N, D), jnp.float32),
        ],
    )(degp, degp, x)


def _mid_body(p0_ref, p1_ref, dv_ref, w1_ref, b1_ref, w2_ref, t_ref):
    dv = dv_ref[...]
    z = (p0_ref[...] + p1_ref[...]) * dv
    h = jnp.dot(z.astype(jnp.bfloat16), w1_ref[...],
                preferred_element_type=jnp.float32) + b1_ref[...]
    h = jnp.maximum(h, 0.0)
    t_ref[...] = jnp.dot(h.astype(jnp.bfloat16), w2_ref[...],
                         preferred_element_type=jnp.float32) * dv


def _tc_mid(p, dv, W1, b1, W2, N, D, H, R):
    nb = N // R
    return pl.pallas_call(
        _mid_body,
        grid=(nb,),
        in_specs=[
            pl.BlockSpec((R, D), lambda i: (i, 0)),
            pl.BlockSpec((R, D), lambda i: (nb + i, 0)),
            pl.BlockSpec((R, D), lambda i: (i, 0)),
            pl.BlockSpec((D, H), lambda i: (0, 0)),
            pl.BlockSpec((1, H), lambda i: (0, 0)),
            pl.BlockSpec((H, D), lambda i: (0, 0)),
        ],
        out_specs=pl.BlockSpec((R, D), lambda i: (i, 0)),
        out_shape=jax.ShapeDtypeStruct((N, D), jnp.float32),
    )(p, p, dv, W1, b1.reshape(1, H), W2)


def _fin_body(q0_ref, q1_ref, dv_ref, b2_ref, o_ref):
    o_ref[...] = (q0_ref[...] + q1_ref[...]) * dv_ref[...] + b2_ref[...]


def _tc_fin(q, dv, b2, N, D, R):
    nb = N // R
    return pl.pallas_call(
        _fin_body,
        grid=(nb,),
        in_specs=[
            pl.BlockSpec((R, D), lambda i: (i, 0)),
            pl.BlockSpec((R, D), lambda i: (nb + i, 0)),
            pl.BlockSpec((R, D), lambda i: (i, 0)),
            pl.BlockSpec((1, D), lambda i: (0, 0)),
        ],
        out_specs=pl.BlockSpec((R, D), lambda i: (i, 0)),
        out_shape=jax.ShapeDtypeStruct((N, D), jnp.float32),
    )(q, q, dv, b2.reshape(1, D))


def kernel(x, edge_index, W1, b1, W2, b2):
    N, D = x.shape
    E = edge_index.shape[1]
    H = W1.shape[1]
    CH = 80
    CHA = 80
    R = 5000

    ef = edge_index.reshape(-1)
    zb = jnp.zeros((1000, D), jnp.float32)
    degp = _sc_deg(ef, N, E, CH)
    xs, dv = _tc_prep(degp, x, N, D, R)
    p = _sc_agg(xs, ef, zb, N, E, D, CHA)
    t = _tc_mid(p, dv, W1.astype(jnp.bfloat16), b1, W2.astype(jnp.bfloat16),
                N, D, H, R)
    q = _sc_agg(t, ef, zb, N, E, D, CHA)
    return _tc_fin(q, dv, b2, N, D, R)


# speedup vs baseline: 1.0206x; 1.0060x over previous
import functools

import jax
import jax.numpy as jnp
from jax import lax
from jax.experimental import pallas as pl
from jax.experimental.pallas import tpu as pltpu
from jax.experimental.pallas import tpu_sc as plsc

NC = 2
NS = 16
LN = 16


def _mesh():
    return plsc.VectorSubcoreMesh(
        core_axis_name="c", subcore_axis_name="s", num_cores=NC, num_subcores=NS
    )


def _deg_body(N, E, CH, NR, ef_hbm, out_hbm, acc, zbuf, obuf, ci, *ss):
    c = lax.axis_index("c")
    s = lax.axis_index("s")
    ct = N // 1000
    r0 = s * 1000

    def zrow(i, carry):
        zbuf[i, pl.ds(0, LN)] = jnp.zeros((LN,), jnp.float32)
        return carry

    lax.fori_loop(0, zbuf.shape[0], zrow, 0)

    def orow(i, carry):
        obuf[i, pl.ds(0, LN)] = jnp.ones((LN,), jnp.float32)
        return carry

    lax.fori_loop(0, CH, orow, 0)

    ec = E // NC
    et = ec // NS
    ni = et // CH
    base0 = c * ec + s * et
    pltpu.async_copy(ef_hbm.at[pl.ds(E + base0, et)], ci, ss[0])

    @pl.when(s < ct)
    def _zero():
        pltpu.sync_copy(zbuf, acc.at[pl.ds(r0, 1000)])

    pltpu.make_async_copy(ef_hbm.at[pl.ds(E + base0, et)], ci, ss[0]).wait()
    plsc.subcore_barrier()

    def start_scatter(b, ch):
        pltpu.async_copy(obuf, acc.at[ci.at[pl.ds(ch * CH, CH)]], ss[b], add=True)

    def wait_scatter(b):
        pltpu.make_async_copy(obuf, acc.at[ci.at[pl.ds(0, CH)]], ss[b]).wait()

    def outer(o, carry):
        for b in range(NR):
            ch = o * NR + b

            @pl.when(ch >= NR)
            def _drain():
                wait_scatter(b)

            start_scatter(b, ch)
        return carry

    lax.fori_loop(0, ni // NR, outer, 0)
    for ch in range((ni // NR) * NR, ni):
        b = ch % NR
        if ch >= NR:
            wait_scatter(b)
        start_scatter(b, ch)
    for b in range(NR):
        wait_scatter(b)

    plsc.subcore_barrier()

    @pl.when(s < ct)
    def _copyout():
        pltpu.sync_copy(acc.at[pl.ds(r0, 1000)], out_hbm.at[pl.ds(c * N + r0, 1000)])


def _sc_deg(ef, N, E, CH):
    NR = 4
    et = E // NC // NS
    body = functools.partial(_deg_body, N, E, CH, NR)
    return pl.kernel(
        body,
        out_type=jax.ShapeDtypeStruct((NC * N, LN), jnp.float32),
        mesh=_mesh(),
        compiler_params=pltpu.CompilerParams(use_tc_tiling_on_sc=False),
        scratch_types=[
            pltpu.VMEM_SHARED((N, LN), jnp.float32),
            pltpu.VMEM((1000, LN), jnp.float32),
            pltpu.VMEM((CH, LN), jnp.float32),
            pltpu.VMEM((et,), jnp.int32),
        ]
        + [pltpu.SemaphoreType.DMA] * NR,
    )(ef)


NB = 3


def _agg_body(N, E, D, CH, vals_hbm, ef_hbm, z_hbm, out_hbm, acc, ri, ci, *bufs):
    gb = bufs[0:NB]
    sm = bufs[NB:2 * NB]
    ss = bufs[2 * NB:3 * NB]
    c = lax.axis_index("c")
    s = lax.axis_index("s")
    ct = N // 1000
    r0 = s * 1000

    ec = E // NC
    et = ec // NS
    ni = et // CH
    base0 = c * ec + s * et

    pltpu.async_copy(ef_hbm.at[pl.ds(base0, et)], ri, sm[0])
    pltpu.async_copy(ef_hbm.at[pl.ds(E + base0, et)], ci, sm[1])

    @pl.when(s < ct)
    def _zero():
        pltpu.sync_copy(z_hbm, acc.at[pl.ds(r0, 1000)])

    def start_gather(b, ch):
        pltpu.async_copy(vals_hbm.at[ri.at[pl.ds(ch * CH, CH)]], gb[b], sm[b])

    def wait_gather(b):
        pltpu.make_async_copy(vals_hbm.at[ri.at[pl.ds(0, CH)]], gb[b], sm[b]).wait()

    def start_scatter(b, ch):
        pltpu.async_copy(gb[b], acc.at[ci.at[pl.ds(ch * CH, CH)]], ss[b], add=True)

    def wait_scatter(b):
        pltpu.make_async_copy(gb[b], acc.at[ci.at[pl.ds(0, CH)]], ss[b]).wait()

    pltpu.make_async_copy(ef_hbm.at[pl.ds(base0, et)], ri, sm[0]).wait()
    pltpu.make_async_copy(ef_hbm.at[pl.ds(E + base0, et)], ci, sm[1]).wait()
    for b in range(NB - 1):
        start_gather(b, b)
    plsc.subcore_barrier()

    def outer(o, carry):
        for b in range(NB):
            ch = o * NB + b
            pb = (b + NB - 1) % NB
            wait_gather(b)
            start_scatter(b, ch)

            @pl.when(ch + NB - 1 < ni)
            def _prefetch():
                @pl.when(ch >= 1)
                def _drain():
                    wait_scatter(pb)

                start_gather(pb, ch + NB - 1)
        return carry

    lax.fori_loop(0, ni // NB, outer, 0)
    for ch in range((ni // NB) * NB, ni):
        b = ch % NB
        pb = (b + NB - 1) % NB
        wait_gather(b)
        start_scatter(b, ch)
        if ch + NB - 1 < ni:
            wait_scatter(pb)
            start_gather(pb, ch + NB - 1)
    for b in range(NB):
        wait_scatter(b)

    plsc.subcore_barrier()

    @pl.when(s < ct)
    def _copyout():
        pltpu.sync_copy(acc.at[pl.ds(r0, 1000)], out_hbm.at[pl.ds(c * N + r0, 1000)])


def _sc_agg(vals, ef, zb, N, E, D, CH):
    et = E // NC // NS
    body = functools.partial(_agg_body, N, E, D, CH)
    return pl.kernel(
        body,
        out_type=jax.ShapeDtypeStruct((NC * N, D), jnp.float32),
        mesh=_mesh(),
        compiler_params=pltpu.CompilerParams(use_tc_tiling_on_sc=False),
        scratch_types=[
            pltpu.VMEM_SHARED((N, D), jnp.float32),
            pltpu.VMEM((et,), jnp.int32),
            pltpu.VMEM((et,), jnp.int32),
        ]
        + [pltpu.VMEM((CH, D), jnp.float32)] * NB
        + [pltpu.SemaphoreType.DMA] * (2 * NB),
    )(vals, ef, zb)


def _prep_body(deg0_ref, deg1_ref, x_ref, xs_ref, dv_ref):
    deg = deg0_ref[:, 0:1] + deg1_ref[:, 0:1]
    dinv = jnp.where(deg > 0, lax.rsqrt(deg), 0.0)
    dv = jnp.broadcast_to(dinv, x_ref.shape)
    dv_ref[...] = dv
    xs_ref[...] = x_ref[...] * dv


def _tc_prep(degp, x, N, D, R):
    nb = N // R
    return pl.pallas_call(
        _prep_body,
        grid=(nb,),
        in_specs=[
            pl.BlockSpec((R, LN), lambda i: (i, 0)),
            pl.BlockSpec((R, LN), lambda i: (nb + i, 0)),
            pl.BlockSpec((R, D), lambda i: (i, 0)),
        ],
        out_specs=[
            pl.BlockSpec((R, D), lambda i: (i, 0)),
            pl.BlockSpec((R, D), lambda i: (i, 0)),
        ],
        out_shape=[
            jax.ShapeDtypeStruct((N, D), jnp.float32),
            jax.ShapeDtypeStruct((N, D), jnp.float32),
        ],
    )(degp, degp, x)


def _mid_body(p0_ref, p1_ref, dv_ref, w1_ref, b1_ref, w2_ref, t_ref):
    dv = dv_ref[...]
    z = (p0_ref[...] + p1_ref[...]) * dv
    h = jnp.dot(z, w1_ref[...], preferred_element_type=jnp.float32) + b1_ref[...]
    h = jnp.maximum(h, 0.0)
    t_ref[...] = jnp.dot(h, w2_ref[...], preferred_element_type=jnp.float32) * dv


def _tc_mid(p, dv, W1, b1, W2, N, D, H, R):
    nb = N // R
    return pl.pallas_call(
        _mid_body,
        grid=(nb,),
        in_specs=[
            pl.BlockSpec((R, D), lambda i: (i, 0)),
            pl.BlockSpec((R, D), lambda i: (nb + i, 0)),
            pl.BlockSpec((R, D), lambda i: (i, 0)),
            pl.BlockSpec((D, H), lambda i: (0, 0)),
            pl.BlockSpec((1, H), lambda i: (0, 0)),
            pl.BlockSpec((H, D), lambda i: (0, 0)),
        ],
        out_specs=pl.BlockSpec((R, D), lambda i: (i, 0)),
        out_shape=jax.ShapeDtypeStruct((N, D), jnp.float32),
    )(p, p, dv, W1, b1.reshape(1, H), W2)


def _fin_body(q0_ref, q1_ref, dv_ref, b2_ref, o_ref):
    o_ref[...] = (q0_ref[...] + q1_ref[...]) * dv_ref[...] + b2_ref[...]


def _tc_fin(q, dv, b2, N, D, R):
    nb = N // R
    return pl.pallas_call(
        _fin_body,
        grid=(nb,),
        in_specs=[
            pl.BlockSpec((R, D), lambda i: (i, 0)),
            pl.BlockSpec((R, D), lambda i: (nb + i, 0)),
            pl.BlockSpec((R, D), lambda i: (i, 0)),
            pl.BlockSpec((1, D), lambda i: (0, 0)),
        ],
        out_specs=pl.BlockSpec((R, D), lambda i: (i, 0)),
        out_shape=jax.ShapeDtypeStruct((N, D), jnp.float32),
    )(q, q, dv, b2.reshape(1, D))


def kernel(x, edge_index, W1, b1, W2, b2):
    N, D = x.shape
    E = edge_index.shape[1]
    H = W1.shape[1]
    CH = 80
    CHA = 80
    R = 5000

    ef = edge_index.reshape(-1)
    zb = jnp.zeros((1000, D), jnp.float32)
    degp = _sc_deg(ef, N, E, CH)
    xs, dv = _tc_prep(degp, x, N, D, R)
    p = _sc_agg(xs, ef, zb, N, E, D, CHA)
    t = _tc_mid(p, dv, W1, b1, W2, N, D, H, R)
    q = _sc_agg(t, ef, zb, N, E, D, CHA)
    return _tc_fin(q, dv, b2, N, D, R)


# speedup vs baseline: 1.0240x; 1.0033x over previous
import functools

import jax
import jax.numpy as jnp
from jax import lax
from jax.experimental import pallas as pl
from jax.experimental.pallas import tpu as pltpu
from jax.experimental.pallas import tpu_sc as plsc

NC = 2
NS = 16
LN = 16


def _mesh():
    return plsc.VectorSubcoreMesh(
        core_axis_name="c", subcore_axis_name="s", num_cores=NC, num_subcores=NS
    )


def _deg_body(N, E, CH, NR, ef_hbm, out_hbm, acc, zbuf, obuf, ci, *ss):
    c = lax.axis_index("c")
    s = lax.axis_index("s")
    ct = N // 1000
    r0 = s * 1000

    ec = E // NC
    et = ec // NS
    ni = et // CH
    base0 = c * ec + s * et
    pltpu.async_copy(ef_hbm.at[pl.ds(E + base0, et)], ci, ss[0])

    def zrow(i, carry):
        zbuf[i, pl.ds(0, LN)] = jnp.zeros((LN,), jnp.float32)
        return carry

    lax.fori_loop(0, zbuf.shape[0], zrow, 0)

    def orow(i, carry):
        obuf[i, pl.ds(0, LN)] = jnp.ones((LN,), jnp.float32)
        return carry

    lax.fori_loop(0, CH, orow, 0)

    @pl.when(s < ct)
    def _zero():
        pltpu.sync_copy(zbuf, acc.at[pl.ds(r0, 1000)])

    pltpu.make_async_copy(ef_hbm.at[pl.ds(E + base0, et)], ci, ss[0]).wait()
    plsc.subcore_barrier()

    def start_scatter(b, ch):
        pltpu.async_copy(obuf, acc.at[ci.at[pl.ds(ch * CH, CH)]], ss[b], add=True)

    def wait_scatter(b):
        pltpu.make_async_copy(obuf, acc.at[ci.at[pl.ds(0, CH)]], ss[b]).wait()

    def outer(o, carry):
        for b in range(NR):
            ch = o * NR + b

            @pl.when(ch >= NR)
            def _drain():
                wait_scatter(b)

            start_scatter(b, ch)
        return carry

    lax.fori_loop(0, ni // NR, outer, 0)
    for ch in range((ni // NR) * NR, ni):
        b = ch % NR
        if ch >= NR:
            wait_scatter(b)
        start_scatter(b, ch)
    for b in range(NR):
        wait_scatter(b)

    plsc.subcore_barrier()

    @pl.when(s < ct)
    def _copyout():
        pltpu.sync_copy(acc.at[pl.ds(r0, 1000)], out_hbm.at[pl.ds(c * N + r0, 1000)])


def _sc_deg(ef, N, E, CH):
    NR = 4
    et = E // NC // NS
    body = functools.partial(_deg_body, N, E, CH, NR)
    return pl.kernel(
        body,
        out_type=jax.ShapeDtypeStruct((NC * N, LN), jnp.float32),
        mesh=_mesh(),
        compiler_params=pltpu.CompilerParams(use_tc_tiling_on_sc=False),
        scratch_types=[
            pltpu.VMEM_SHARED((N, LN), jnp.float32),
            pltpu.VMEM((1000, LN), jnp.float32),
            pltpu.VMEM((CH, LN), jnp.float32),
            pltpu.VMEM((et,), jnp.int32),
        ]
        + [pltpu.SemaphoreType.DMA] * NR,
    )(ef)


NB = 3


def _agg_body(N, E, D, CH, vals_hbm, ef_hbm, z_hbm, out_hbm, acc, ri, ci, *bufs):
    gb = bufs[0:NB]
    sm = bufs[NB:2 * NB]
    ss = bufs[2 * NB:3 * NB]
    c = lax.axis_index("c")
    s = lax.axis_index("s")
    ct = N // 1000
    r0 = s * 1000

    ec = E // NC
    et = ec // NS
    ni = et // CH
    base0 = c * ec + s * et

    pltpu.async_copy(ef_hbm.at[pl.ds(base0, et)], ri, sm[0])
    pltpu.async_copy(ef_hbm.at[pl.ds(E + base0, et)], ci, sm[1])

    @pl.when(s < ct)
    def _zero():
        pltpu.sync_copy(z_hbm, acc.at[pl.ds(r0, 1000)])

    def start_gather(b, ch):
        pltpu.async_copy(vals_hbm.at[ri.at[pl.ds(ch * CH, CH)]], gb[b], sm[b])

    def wait_gather(b):
        pltpu.make_async_copy(vals_hbm.at[ri.at[pl.ds(0, CH)]], gb[b], sm[b]).wait()

    def start_scatter(b, ch):
        pltpu.async_copy(gb[b], acc.at[ci.at[pl.ds(ch * CH, CH)]], ss[b], add=True)

    def wait_scatter(b):
        pltpu.make_async_copy(gb[b], acc.at[ci.at[pl.ds(0, CH)]], ss[b]).wait()

    pltpu.make_async_copy(ef_hbm.at[pl.ds(base0, et)], ri, sm[0]).wait()
    pltpu.make_async_copy(ef_hbm.at[pl.ds(E + base0, et)], ci, sm[1]).wait()
    for b in range(NB - 1):
        start_gather(b, b)
    plsc.subcore_barrier()

    def outer(o, carry):
        for b in range(NB):
            ch = o * NB + b
            pb = (b + NB - 1) % NB
            wait_gather(b)
            start_scatter(b, ch)

            @pl.when(ch + NB - 1 < ni)
            def _prefetch():
                @pl.when(ch >= 1)
                def _drain():
                    wait_scatter(pb)

                start_gather(pb, ch + NB - 1)
        return carry

    lax.fori_loop(0, ni // NB, outer, 0)
    for ch in range((ni // NB) * NB, ni):
        b = ch % NB
        pb = (b + NB - 1) % NB
        wait_gather(b)
        start_scatter(b, ch)
        if ch + NB - 1 < ni:
            wait_scatter(pb)
            start_gather(pb, ch + NB - 1)
    for b in range(NB):
        wait_scatter(b)

    plsc.subcore_barrier()

    @pl.when(s < ct)
    def _copyout():
        pltpu.sync_copy(acc.at[pl.ds(r0, 1000)], out_hbm.at[pl.ds(c * N + r0, 1000)])


def _sc_agg(vals, ef, zb, N, E, D, CH):
    et = E // NC // NS
    body = functools.partial(_agg_body, N, E, D, CH)
    return pl.kernel(
        body,
        out_type=jax.ShapeDtypeStruct((NC * N, D), jnp.float32),
        mesh=_mesh(),
        compiler_params=pltpu.CompilerParams(use_tc_tiling_on_sc=False),
        scratch_types=[
            pltpu.VMEM_SHARED((N, D), jnp.float32),
            pltpu.VMEM((et,), jnp.int32),
            pltpu.VMEM((et,), jnp.int32),
        ]
        + [pltpu.VMEM((CH, D), jnp.float32)] * NB
        + [pltpu.SemaphoreType.DMA] * (2 * NB),
    )(vals, ef, zb)


def _prep_body(deg0_ref, deg1_ref, x_ref, xs_ref, dv_ref):
    deg = deg0_ref[:, 0:1] + deg1_ref[:, 0:1]
    dinv = jnp.where(deg > 0, lax.rsqrt(deg), 0.0)
    dv = jnp.broadcast_to(dinv, x_ref.shape)
    dv_ref[...] = dv
    xs_ref[...] = x_ref[...] * dv


def _tc_prep(degp, x, N, D, R):
    nb = N // R
    return pl.pallas_call(
        _prep_body,
        grid=(nb,),
        in_specs=[
            pl.BlockSpec((R, LN), lambda i: (i, 0)),
            pl.BlockSpec((R, LN), lambda i: (nb + i, 0)),
            pl.BlockSpec((R, D), lambda i: (i, 0)),
        ],
        out_specs=[
            pl.BlockSpec((R, D), lambda i: (i, 0)),
            pl.BlockSpec((R, D), lambda i: (i, 0)),
        ],
        out_shape=[
            jax.ShapeDtypeStruct((N, D), jnp.float32),
            jax.ShapeDtypeStruct((N, D), jnp.float32),
        ],
    )(degp, degp, x)


def _mid_body(p0_ref, p1_ref, dv_ref, w1_ref, b1_ref, w2_ref, t_ref):
    dv = dv_ref[...]
    z = (p0_ref[...] + p1_ref[...]) * dv
    h = jnp.dot(z, w1_ref[...], preferred_element_type=jnp.float32) + b1_ref[...]
    h = jnp.maximum(h, 0.0)
    t_ref[...] = jnp.dot(h, w2_ref[...], preferred_element_type=jnp.float32) * dv


def _tc_mid(p, dv, W1, b1, W2, N, D, H, R):
    nb = N // R
    return pl.pallas_call(
        _mid_body,
        grid=(nb,),
        in_specs=[
            pl.BlockSpec((R, D), lambda i: (i, 0)),
            pl.BlockSpec((R, D), lambda i: (nb + i, 0)),
            pl.BlockSpec((R, D), lambda i: (i, 0)),
            pl.BlockSpec((D, H), lambda i: (0, 0)),
            pl.BlockSpec((1, H), lambda i: (0, 0)),
            pl.BlockSpec((H, D), lambda i: (0, 0)),
        ],
        out_specs=pl.BlockSpec((R, D), lambda i: (i, 0)),
        out_shape=jax.ShapeDtypeStruct((N, D), jnp.float32),
    )(p, p, dv, W1, b1.reshape(1, H), W2)


def _fin_body(q0_ref, q1_ref, dv_ref, b2_ref, o_ref):
    o_ref[...] = (q0_ref[...] + q1_ref[...]) * dv_ref[...] + b2_ref[...]


def _tc_fin(q, dv, b2, N, D, R):
    nb = N // R
    return pl.pallas_call(
        _fin_body,
        grid=(nb,),
        in_specs=[
            pl.BlockSpec((R, D), lambda i: (i, 0)),
            pl.BlockSpec((R, D), lambda i: (nb + i, 0)),
            pl.BlockSpec((R, D), lambda i: (i, 0)),
            pl.BlockSpec((1, D), lambda i: (0, 0)),
        ],
        out_specs=pl.BlockSpec((R, D), lambda i: (i, 0)),
        out_shape=jax.ShapeDtypeStruct((N, D), jnp.float32),
    )(q, q, dv, b2.reshape(1, D))


def kernel(x, edge_index, W1, b1, W2, b2):
    N, D = x.shape
    E = edge_index.shape[1]
    H = W1.shape[1]
    CH = 80
    CHA = 80
    R = 5000

    ef = edge_index.reshape(-1)
    zb = jnp.zeros((1000, D), jnp.float32)
    degp = _sc_deg(ef, N, E, CH)
    xs, dv = _tc_prep(degp, x, N, D, R)
    p = _sc_agg(xs, ef, zb, N, E, D, CHA)
    t = _tc_mid(p, dv, W1, b1, W2, N, D, H, R)
    q = _sc_agg(t, ef, zb, N, E, D, CHA)
    return _tc_fin(q, dv, b2, N, D, R)
